# baseline (device time: 15823 ns/iter reference)
import jax
import jax.numpy as jnp
from jax import lax
from jax.experimental import pallas as pl
from jax.experimental.pallas import tpu as pltpu

N_CHUNKS = 4
WIRE_DTYPE = jnp.int8
WIRE_SCALE = 128.0 / 127.0


def kernel(x, dy):
    m, d = x.shape
    _, f = dy.shape
    half = d // 2
    fh = f // 2
    rows = half // N_CHUNKS

    def body(x_ref, dy_ref, out_ref,
             x_send_buf, x_recv_buf, z_recv_buf,
             x_send_sems, x_recv_sems, fwd_send_sems, z_recv_sems):
        my_x = lax.axis_index("x")
        my_y = lax.axis_index("y")
        my_z = lax.axis_index("z")
        r = my_z % 2
        x_peer = (1 - my_x, my_y, my_z)
        z_partner = (my_x, my_y, my_z + 1 - 2 * r)

        barrier_sem = pltpu.get_barrier_semaphore()
        for nbr in (x_peer, z_partner):
            pl.semaphore_signal(
                barrier_sem, inc=1, device_id=nbr,
                device_id_type=pl.DeviceIdType.MESH,
            )
        pl.semaphore_wait(barrier_sem, 2)

        xv = x_ref[...].astype(jnp.bfloat16)
        dyv = dy_ref[...].astype(jnp.bfloat16)
        is_x0 = my_x == 0
        is_r0 = r == 0
        x_keep = jnp.where(is_x0, xv[:, :half], xv[:, half:])
        x_send = jnp.where(is_x0, xv[:, half:], xv[:, :half])
        dy_r = jnp.where(is_r0, dyv[:, :fh], dyv[:, fh:])

        x_rdmas = []
        fwd_rdmas = []
        for c in range(N_CHUNKS):
            sl = pl.ds(c * rows, rows)
            x_rdmas.append(pltpu.make_async_remote_copy(
                src_ref=x_send_buf.at[sl, :],
                dst_ref=x_recv_buf.at[sl, :],
                send_sem=x_send_sems.at[c],
                recv_sem=x_recv_sems.at[c],
                device_id=x_peer,
                device_id_type=pl.DeviceIdType.MESH,
            ))
            fwd_rdmas.append(pltpu.make_async_remote_copy(
                src_ref=x_recv_buf.at[sl, :],
                dst_ref=z_recv_buf.at[sl, :],
                send_sem=fwd_send_sems.at[c],
                recv_sem=z_recv_sems.at[c],
                device_id=z_partner,
                device_id_type=pl.DeviceIdType.MESH,
            ))

        for c in range(N_CHUNKS):
            sp = lax.dot_general(
                x_send[:, c * rows:(c + 1) * rows], dy_r,
                (((0,), (0,)), ((), ())),
                preferred_element_type=jnp.float32,
            )
            q = jnp.clip(jnp.round(sp * (1.0 / WIRE_SCALE)), -127.0, 127.0)
            x_send_buf[c * rows:(c + 1) * rows, :] = q.astype(WIRE_DTYPE)
            x_rdmas[c].start()

        keep0 = lax.dot_general(
            x_keep, dyv[:, :fh], (((0,), (0,)), ((), ())),
            preferred_element_type=jnp.float32,
        )
        for c in range(N_CHUNKS // 2):
            x_rdmas[c].wait_recv()
            fwd_rdmas[c].start()

        keep1 = lax.dot_general(
            x_keep, dyv[:, fh:], (((0,), (0,)), ((), ())),
            preferred_element_type=jnp.float32,
        )
        for c in range(N_CHUNKS // 2, N_CHUNKS):
            x_rdmas[c].wait_recv()
            fwd_rdmas[c].start()

        for rd in fwd_rdmas:
            rd.wait_recv()

        xr = x_recv_buf[...].astype(jnp.float32) * WIRE_SCALE
        zr = z_recv_buf[...].astype(jnp.float32) * WIRE_SCALE
        out_ref[:, :fh] = keep0 + jnp.where(is_r0, xr, zr)
        out_ref[:, fh:] = keep1 + jnp.where(is_r0, zr, xr)

        for rd in x_rdmas:
            rd.wait_send()
        for rd in fwd_rdmas:
            rd.wait_send()

    return pl.pallas_call(
        body,
        out_shape=jax.ShapeDtypeStruct((half, f), jnp.float32),
        in_specs=[
            pl.BlockSpec(memory_space=pltpu.VMEM),
            pl.BlockSpec(memory_space=pltpu.VMEM),
        ],
        out_specs=pl.BlockSpec(memory_space=pltpu.VMEM),
        scratch_shapes=[
            pltpu.VMEM((half, fh), WIRE_DTYPE),
            pltpu.VMEM((half, fh), WIRE_DTYPE),
            pltpu.VMEM((half, fh), WIRE_DTYPE),
            pltpu.SemaphoreType.DMA((N_CHUNKS,)),
            pltpu.SemaphoreType.DMA((N_CHUNKS,)),
            pltpu.SemaphoreType.DMA((N_CHUNKS,)),
            pltpu.SemaphoreType.DMA((N_CHUNKS,)),
        ],
        compiler_params=pltpu.CompilerParams(collective_id=0),
    )(x, dy)


# device time: 15232 ns/iter; 1.0388x vs baseline; 1.0388x over previous
import jax
import jax.numpy as jnp
from jax import lax
from jax.experimental import pallas as pl
from jax.experimental.pallas import tpu as pltpu

N_CHUNKS = 4
WIRE_DTYPE = jnp.int8
WIRE_SCALE = 128.0 / 127.0


def _quant(v):
    return jnp.clip(jnp.round(v * (1.0 / WIRE_SCALE)), -127.0, 127.0).astype(
        WIRE_DTYPE
    )


def kernel(x, dy):
    m, d = x.shape
    _, f = dy.shape
    half = d // 2
    fh = f // 2
    rows = half // N_CHUNKS

    def body(x_ref, dy_ref, out_ref,
             x_send_buf, x_recv_buf, z_recv_buf, yk_send_buf, yk_recv_buf,
             x_send_sems, x_recv_sems, fwd_send_sems, z_recv_sems, yk_sems):
        my_x = lax.axis_index("x")
        my_y = lax.axis_index("y")
        my_z = lax.axis_index("z")
        r = my_z % 2
        x_peer = (1 - my_x, my_y, my_z)
        y_partner = (my_x, 1 - my_y, my_z)
        z_partner = (my_x, my_y, my_z + 1 - 2 * r)

        barrier_sem = pltpu.get_barrier_semaphore()
        for nbr in (x_peer, y_partner, z_partner):
            pl.semaphore_signal(
                barrier_sem, inc=1, device_id=nbr,
                device_id_type=pl.DeviceIdType.MESH,
            )
        pl.semaphore_wait(barrier_sem, 3)

        xv = x_ref[...].astype(jnp.bfloat16)
        dyv = dy_ref[...].astype(jnp.bfloat16)
        is_x0 = my_x == 0
        is_r0 = r == 0
        is_h0 = my_y == 0
        x_keep = jnp.where(is_x0, xv[:, :half], xv[:, half:])
        x_send = jnp.where(is_x0, xv[:, half:], xv[:, :half])
        dy_r = jnp.where(is_r0, dyv[:, :fh], dyv[:, fh:])
        dy_h = jnp.where(is_h0, dyv[:, :fh], dyv[:, fh:])

        x_rdmas = []
        fwd_rdmas = []
        for c in range(N_CHUNKS):
            sl = pl.ds(c * rows, rows)
            x_rdmas.append(pltpu.make_async_remote_copy(
                src_ref=x_send_buf.at[sl, :],
                dst_ref=x_recv_buf.at[sl, :],
                send_sem=x_send_sems.at[c],
                recv_sem=x_recv_sems.at[c],
                device_id=x_peer,
                device_id_type=pl.DeviceIdType.MESH,
            ))
            fwd_rdmas.append(pltpu.make_async_remote_copy(
                src_ref=x_recv_buf.at[sl, :],
                dst_ref=z_recv_buf.at[sl, :],
                send_sem=fwd_send_sems.at[c],
                recv_sem=z_recv_sems.at[c],
                device_id=z_partner,
                device_id_type=pl.DeviceIdType.MESH,
            ))
        yk_rdma = pltpu.make_async_remote_copy(
            src_ref=yk_send_buf,
            dst_ref=yk_recv_buf,
            send_sem=yk_sems.at[0],
            recv_sem=yk_sems.at[1],
            device_id=y_partner,
            device_id_type=pl.DeviceIdType.MESH,
        )

        for c in range(N_CHUNKS):
            sp = lax.dot_general(
                x_send[:, c * rows:(c + 1) * rows], dy_r,
                (((0,), (0,)), ((), ())),
                preferred_element_type=jnp.float32,
            )
            x_send_buf[c * rows:(c + 1) * rows, :] = _quant(sp)
            x_rdmas[c].start()

        keep_mine = lax.dot_general(
            x_keep, dy_h, (((0,), (0,)), ((), ())),
            preferred_element_type=jnp.float32,
        )
        yk_send_buf[...] = _quant(keep_mine)
        yk_rdma.start()

        for c in range(N_CHUNKS):
            x_rdmas[c].wait_recv()
            fwd_rdmas[c].start()

        yk_rdma.wait_recv()
        for rd in fwd_rdmas:
            rd.wait_recv()

        xr = x_recv_buf[...].astype(jnp.float32) * WIRE_SCALE
        zr = z_recv_buf[...].astype(jnp.float32) * WIRE_SCALE
        ykr = yk_recv_buf[...].astype(jnp.float32) * WIRE_SCALE
        out_ref[:, :fh] = (
            jnp.where(is_h0, keep_mine, ykr) + jnp.where(is_r0, xr, zr)
        )
        out_ref[:, fh:] = (
            jnp.where(is_h0, ykr, keep_mine) + jnp.where(is_r0, zr, xr)
        )

        for rd in x_rdmas:
            rd.wait_send()
        for rd in fwd_rdmas:
            rd.wait_send()
        yk_rdma.wait_send()

    return pl.pallas_call(
        body,
        out_shape=jax.ShapeDtypeStruct((half, f), jnp.float32),
        in_specs=[
            pl.BlockSpec(memory_space=pltpu.VMEM),
            pl.BlockSpec(memory_space=pltpu.VMEM),
        ],
        out_specs=pl.BlockSpec(memory_space=pltpu.VMEM),
        scratch_shapes=[
            pltpu.VMEM((half, fh), WIRE_DTYPE),
            pltpu.VMEM((half, fh), WIRE_DTYPE),
            pltpu.VMEM((half, fh), WIRE_DTYPE),
            pltpu.VMEM((half, fh), WIRE_DTYPE),
            pltpu.VMEM((half, fh), WIRE_DTYPE),
            pltpu.SemaphoreType.DMA((N_CHUNKS,)),
            pltpu.SemaphoreType.DMA((N_CHUNKS,)),
            pltpu.SemaphoreType.DMA((N_CHUNKS,)),
            pltpu.SemaphoreType.DMA((N_CHUNKS,)),
            pltpu.SemaphoreType.DMA((2,)),
        ],
        compiler_params=pltpu.CompilerParams(collective_id=0),
    )(x, dy)
